# baseline (speedup 1.0000x reference)
import functools

import jax
import jax.numpy as jnp
from jax import lax
from jax.experimental import pallas as pl
from jax.experimental.pallas import tpu as pltpu
from jax.experimental.pallas import tpu_sc as plsc

N_NODES = 10000
N_EDGES = 320000
D = 128

NUM_CORES = 2
NUM_SUBCORES = 16
NUM_WORKERS = NUM_CORES * NUM_SUBCORES
LANES = 16

CHUNK = 80
N_PAD = 10112
ROWS_PER_TILE = N_PAD // NUM_SUBCORES
FAST_CID = 0
CPB_FAST = 164
CPB_SLOW = 88
E_PAD = (CPB_FAST + CPB_SLOW) * NUM_SUBCORES * CHUNK
N_ROWS_IDX = E_PAD // CHUNK
NBUF = 4

_SUB = D // LANES
_GRPS = CHUNK // LANES


def _spmm_body(sup_hbm, pidx_hbm, out_hbm,
               ibuf, sbuf, rows, agg, gsem, ssem, isem):
    cid = lax.axis_index("c")
    sid = lax.axis_index("s")
    on_fast = cid == FAST_CID
    nch = jnp.where(on_fast, CPB_FAST, CPB_SLOW)
    g0 = jnp.where(on_fast, sid * CPB_FAST,
                   NUM_SUBCORES * CPB_FAST + sid * CPB_SLOW)

    def _idx_copy(c, ib):
        pltpu.async_copy(pidx_hbm.at[g0 + c], ibuf.at[ib], isem.at[ib])

    def _idx_wait(c, ib):
        pltpu.make_async_copy(pidx_hbm.at[g0 + c], ibuf.at[ib],
                              isem.at[ib]).wait()

    def _gather(c, b):
        pltpu.async_copy(sup_hbm.at[ibuf.at[b, 0]], rows.at[b], gsem.at[b])

    def _gather_wait(b):
        pltpu.make_async_copy(sup_hbm.at[ibuf.at[b, 0]], rows.at[b],
                              gsem.at[b]).wait()

    def _scatter(b):
        pltpu.async_copy(rows.at[b], agg.at[sbuf.at[b]], ssem.at[b],
                         add=True)

    def _scatter_wait(b):
        pltpu.make_async_copy(rows.at[b], agg.at[sbuf.at[b]],
                              ssem.at[b]).wait()

    _idx_copy(0, 0)
    _idx_copy(1, 1)
    _idx_copy(2, 2)

    def _zrow(r, _):
        for j in range(_SUB):
            rows[NBUF - 1, r, pl.ds(j * LANES, LANES)] = jnp.zeros(
                (LANES,), jnp.float32)
        return 0
    lax.fori_loop(0, CHUNK, _zrow, 0)
    _idx_wait(0, 0)
    _gather(0, 0)
    _idx_wait(1, 1)
    _gather(1, 1)
    _NZ = ROWS_PER_TILE // CHUNK
    for k in range(_NZ):
        pltpu.sync_copy(rows.at[NBUF - 1],
                        agg.at[pl.ds(sid * ROWS_PER_TILE + k * CHUNK, CHUNK)])
    pltpu.sync_copy(
        rows.at[NBUF - 1, pl.ds(0, ROWS_PER_TILE - _NZ * CHUNK)],
        agg.at[pl.ds(sid * ROWS_PER_TILE + _NZ * CHUNK,
                     ROWS_PER_TILE - _NZ * CHUNK)])
    plsc.subcore_barrier()

    def _quad(q, _):
        for b in range(NBUF):
            c = q * NBUF + b
            bnn = (b + 2) % NBUF
            bnnn = (b + 3) % NBUF

            @pl.when(c >= 2)
            def _():
                _scatter_wait(bnn)

            @pl.when(c + 2 < nch)
            def _():
                _idx_wait(c + 2, bnn)
                _gather(c + 2, bnn)

            @pl.when(c + 3 < nch)
            def _():
                _idx_copy(c + 3, bnnn)

            _gather_wait(b)

            def _grp(g, _):
                vvi = ibuf[b, 2, pl.ds(g * LANES, LANES)]
                vv = lax.bitcast_convert_type(vvi, jnp.float32)
                for j in range(LANES):
                    bc = lax.gather(
                        vv, jnp.full((LANES, 1), j, jnp.int32),
                        lax.GatherDimensionNumbers(offset_dims=(),
                                                   collapsed_slice_dims=(0,),
                                                   start_index_map=(0,)),
                        (1,), mode=lax.GatherScatterMode.PROMISE_IN_BOUNDS)
                    e = g * LANES + j
                    for k in range(_SUB):
                        sl = pl.ds(k * LANES, LANES)
                        rows[b, e, sl] = rows[b, e, sl] * bc
                return 0
            lax.fori_loop(0, _GRPS, _grp, 0)

            for g in range(_GRPS):
                sl = pl.ds(g * LANES, LANES)
                sbuf[b, sl] = ibuf[b, 1, sl]
            _scatter(b)
        return 0
    lax.fori_loop(0, nch // NBUF, _quad, 0)
    _scatter_wait(NBUF - 2)
    _scatter_wait(NBUF - 1)
    plsc.subcore_barrier()

    w0 = sid * ROWS_PER_TILE
    pltpu.sync_copy(agg.at[pl.ds(w0, ROWS_PER_TILE)],
                    out_hbm.at[cid, pl.ds(w0, ROWS_PER_TILE)])


_spmm = pl.kernel(
    _spmm_body,
    out_type=jax.ShapeDtypeStruct((NUM_CORES, N_PAD, D), jnp.float32),
    mesh=plsc.VectorSubcoreMesh(core_axis_name="c", subcore_axis_name="s",
                                num_cores=NUM_CORES,
                                num_subcores=NUM_SUBCORES),
    scratch_types=[
        pltpu.VMEM((NBUF, 3, CHUNK), jnp.int32),
        pltpu.VMEM((NBUF, CHUNK), jnp.int32),
        pltpu.VMEM((NBUF, CHUNK, D), jnp.float32),
        pltpu.VMEM_SHARED((N_PAD, D), jnp.float32),
        pltpu.SemaphoreType.DMA((NBUF,)),
        pltpu.SemaphoreType.DMA((NBUF,)),
        pltpu.SemaphoreType.DMA((NBUF,)),
    ],
)


_ROWS_BLK = 1000
_GRID = N_NODES // _ROWS_BLK


def _mm_first_body(x_ref, w_ref, o_ref):
    o_ref[...] = jnp.dot(x_ref[...], w_ref[...],
                         preferred_element_type=jnp.float32)


def _mm_mid_body(p_ref, b_ref, w_ref, o_ref):
    h = jnp.maximum(p_ref[0] + p_ref[1] + b_ref[...], 0.0)
    o_ref[...] = jnp.dot(h, w_ref[...], preferred_element_type=jnp.float32)


def _combine_body(p_ref, b_ref, o_ref):
    o_ref[...] = p_ref[0] + p_ref[1] + b_ref[...]


_mm_first = pl.pallas_call(
    _mm_first_body,
    grid=(_GRID,),
    in_specs=[
        pl.BlockSpec((_ROWS_BLK, D), lambda i: (i, 0)),
        pl.BlockSpec((D, D), lambda i: (0, 0)),
    ],
    out_specs=pl.BlockSpec((_ROWS_BLK, D), lambda i: (i, 0)),
    out_shape=jax.ShapeDtypeStruct((N_NODES, D), jnp.float32),
)

_mm_mid = pl.pallas_call(
    _mm_mid_body,
    grid=(_GRID,),
    in_specs=[
        pl.BlockSpec((NUM_CORES, _ROWS_BLK, D), lambda i: (0, i, 0)),
        pl.BlockSpec((1, D), lambda i: (0, 0)),
        pl.BlockSpec((D, D), lambda i: (0, 0)),
    ],
    out_specs=pl.BlockSpec((_ROWS_BLK, D), lambda i: (i, 0)),
    out_shape=jax.ShapeDtypeStruct((N_NODES, D), jnp.float32),
)

_combine = pl.pallas_call(
    _combine_body,
    grid=(_GRID,),
    in_specs=[
        pl.BlockSpec((NUM_CORES, _ROWS_BLK, D), lambda i: (0, i, 0)),
        pl.BlockSpec((1, D), lambda i: (0, 0)),
    ],
    out_specs=pl.BlockSpec((_ROWS_BLK, D), lambda i: (i, 0)),
    out_shape=jax.ShapeDtypeStruct((N_NODES, D), jnp.float32),
)


@jax.jit
def kernel(x, edge_index, adj_values, W1, b1, W2, b2, W3, b3):
    npad = E_PAD - N_EDGES
    src = jnp.pad(edge_index[0].astype(jnp.int32), (0, npad)
                  ).reshape(N_ROWS_IDX, CHUNK)
    dst = jnp.pad(edge_index[1].astype(jnp.int32), (0, npad),
                  constant_values=N_NODES).reshape(N_ROWS_IDX, CHUNK)
    val = lax.bitcast_convert_type(jnp.pad(adj_values, (0, npad)),
                                   jnp.int32).reshape(N_ROWS_IDX, CHUNK)
    pidx = jnp.stack([src, dst, val], axis=1)
    b1r = b1.reshape(1, D)
    b2r = b2.reshape(1, D)
    b3r = b3.reshape(1, D)

    sup = _mm_first(x, W1)
    p = _spmm(sup, pidx)
    sup = _mm_mid(p, b1r, W2)
    p = _spmm(sup, pidx)
    sup = _mm_mid(p, b2r, W3)
    p = _spmm(sup, pidx)
    return _combine(p, b3r)


# speedup vs baseline: 1.0101x; 1.0101x over previous
import functools

import jax
import jax.numpy as jnp
from jax import lax
from jax.experimental import pallas as pl
from jax.experimental.pallas import tpu as pltpu
from jax.experimental.pallas import tpu_sc as plsc

N_NODES = 10000
N_EDGES = 320000
D = 128

NUM_CORES = 2
NUM_SUBCORES = 16
NUM_WORKERS = NUM_CORES * NUM_SUBCORES
LANES = 16

CHUNK = 80
N_PAD = 10112
ROWS_PER_TILE = N_PAD // NUM_SUBCORES
FAST_CID = 0
CPB_FAST = 168
CPB_SLOW = 84
E_PAD = (CPB_FAST + CPB_SLOW) * NUM_SUBCORES * CHUNK
N_ROWS_IDX = E_PAD // CHUNK
NBUF = 4

_SUB = D // LANES
_GRPS = CHUNK // LANES


def _spmm_body(sup_hbm, pidx_hbm, out_hbm,
               ibuf, sbuf, rows, agg, gsem, ssem, isem):
    cid = lax.axis_index("c")
    sid = lax.axis_index("s")
    on_fast = cid == FAST_CID
    nch = jnp.where(on_fast, CPB_FAST, CPB_SLOW)
    g0 = jnp.where(on_fast, sid * CPB_FAST,
                   NUM_SUBCORES * CPB_FAST + sid * CPB_SLOW)

    def _idx_copy(c, ib):
        pltpu.async_copy(pidx_hbm.at[g0 + c], ibuf.at[ib], isem.at[ib])

    def _idx_wait(c, ib):
        pltpu.make_async_copy(pidx_hbm.at[g0 + c], ibuf.at[ib],
                              isem.at[ib]).wait()

    def _gather(c, b):
        pltpu.async_copy(sup_hbm.at[ibuf.at[b, 0]], rows.at[b], gsem.at[b])

    def _gather_wait(b):
        pltpu.make_async_copy(sup_hbm.at[ibuf.at[b, 0]], rows.at[b],
                              gsem.at[b]).wait()

    def _scatter(b):
        pltpu.async_copy(rows.at[b], agg.at[sbuf.at[b]], ssem.at[b],
                         add=True)

    def _scatter_wait(b):
        pltpu.make_async_copy(rows.at[b], agg.at[sbuf.at[b]],
                              ssem.at[b]).wait()

    _idx_copy(0, 0)
    _idx_copy(1, 1)
    _idx_copy(2, 2)

    def _zrow(r, _):
        for j in range(_SUB):
            rows[NBUF - 1, r, pl.ds(j * LANES, LANES)] = jnp.zeros(
                (LANES,), jnp.float32)
        return 0
    lax.fori_loop(0, CHUNK, _zrow, 0)
    _idx_wait(0, 0)
    _gather(0, 0)
    _idx_wait(1, 1)
    _gather(1, 1)
    _NZ = ROWS_PER_TILE // CHUNK
    for k in range(_NZ):
        pltpu.sync_copy(rows.at[NBUF - 1],
                        agg.at[pl.ds(sid * ROWS_PER_TILE + k * CHUNK, CHUNK)])
    pltpu.sync_copy(
        rows.at[NBUF - 1, pl.ds(0, ROWS_PER_TILE - _NZ * CHUNK)],
        agg.at[pl.ds(sid * ROWS_PER_TILE + _NZ * CHUNK,
                     ROWS_PER_TILE - _NZ * CHUNK)])
    plsc.subcore_barrier()

    def _quad(q, _):
        for b in range(NBUF):
            c = q * NBUF + b
            bnn = (b + 2) % NBUF
            bnnn = (b + 3) % NBUF

            @pl.when(c >= 2)
            def _():
                _scatter_wait(bnn)

            @pl.when(c + 2 < nch)
            def _():
                _idx_wait(c + 2, bnn)
                _gather(c + 2, bnn)

            @pl.when(c + 3 < nch)
            def _():
                _idx_copy(c + 3, bnnn)

            _gather_wait(b)

            def _grp(g, _):
                vvi = ibuf[b, 2, pl.ds(g * LANES, LANES)]
                vv = lax.bitcast_convert_type(vvi, jnp.float32)
                for j in range(LANES):
                    bc = lax.gather(
                        vv, jnp.full((LANES, 1), j, jnp.int32),
                        lax.GatherDimensionNumbers(offset_dims=(),
                                                   collapsed_slice_dims=(0,),
                                                   start_index_map=(0,)),
                        (1,), mode=lax.GatherScatterMode.PROMISE_IN_BOUNDS)
                    e = g * LANES + j
                    for k in range(_SUB):
                        sl = pl.ds(k * LANES, LANES)
                        rows[b, e, sl] = rows[b, e, sl] * bc
                return 0
            lax.fori_loop(0, _GRPS, _grp, 0)

            for g in range(_GRPS):
                sl = pl.ds(g * LANES, LANES)
                sbuf[b, sl] = ibuf[b, 1, sl]
            _scatter(b)
        return 0
    lax.fori_loop(0, nch // NBUF, _quad, 0)
    _scatter_wait(NBUF - 2)
    _scatter_wait(NBUF - 1)
    plsc.subcore_barrier()

    w0 = sid * ROWS_PER_TILE
    pltpu.sync_copy(agg.at[pl.ds(w0, ROWS_PER_TILE)],
                    out_hbm.at[cid, pl.ds(w0, ROWS_PER_TILE)])


_spmm = pl.kernel(
    _spmm_body,
    out_type=jax.ShapeDtypeStruct((NUM_CORES, N_PAD, D), jnp.float32),
    mesh=plsc.VectorSubcoreMesh(core_axis_name="c", subcore_axis_name="s",
                                num_cores=NUM_CORES,
                                num_subcores=NUM_SUBCORES),
    scratch_types=[
        pltpu.VMEM((NBUF, 3, CHUNK), jnp.int32),
        pltpu.VMEM((NBUF, CHUNK), jnp.int32),
        pltpu.VMEM((NBUF, CHUNK, D), jnp.float32),
        pltpu.VMEM_SHARED((N_PAD, D), jnp.float32),
        pltpu.SemaphoreType.DMA((NBUF,)),
        pltpu.SemaphoreType.DMA((NBUF,)),
        pltpu.SemaphoreType.DMA((NBUF,)),
    ],
)


_ROWS_BLK = 1000
_GRID = N_NODES // _ROWS_BLK


def _mm_first_body(x_ref, w_ref, o_ref):
    o_ref[...] = jnp.dot(x_ref[...], w_ref[...],
                         preferred_element_type=jnp.float32)


def _mm_mid_body(p_ref, b_ref, w_ref, o_ref):
    h = jnp.maximum(p_ref[0] + p_ref[1] + b_ref[...], 0.0)
    o_ref[...] = jnp.dot(h, w_ref[...], preferred_element_type=jnp.float32)


def _combine_body(p_ref, b_ref, o_ref):
    o_ref[...] = p_ref[0] + p_ref[1] + b_ref[...]


_mm_first = pl.pallas_call(
    _mm_first_body,
    grid=(_GRID,),
    in_specs=[
        pl.BlockSpec((_ROWS_BLK, D), lambda i: (i, 0)),
        pl.BlockSpec((D, D), lambda i: (0, 0)),
    ],
    out_specs=pl.BlockSpec((_ROWS_BLK, D), lambda i: (i, 0)),
    out_shape=jax.ShapeDtypeStruct((N_NODES, D), jnp.float32),
)

_mm_mid = pl.pallas_call(
    _mm_mid_body,
    grid=(_GRID,),
    in_specs=[
        pl.BlockSpec((NUM_CORES, _ROWS_BLK, D), lambda i: (0, i, 0)),
        pl.BlockSpec((1, D), lambda i: (0, 0)),
        pl.BlockSpec((D, D), lambda i: (0, 0)),
    ],
    out_specs=pl.BlockSpec((_ROWS_BLK, D), lambda i: (i, 0)),
    out_shape=jax.ShapeDtypeStruct((N_NODES, D), jnp.float32),
)

_combine = pl.pallas_call(
    _combine_body,
    grid=(_GRID,),
    in_specs=[
        pl.BlockSpec((NUM_CORES, _ROWS_BLK, D), lambda i: (0, i, 0)),
        pl.BlockSpec((1, D), lambda i: (0, 0)),
    ],
    out_specs=pl.BlockSpec((_ROWS_BLK, D), lambda i: (i, 0)),
    out_shape=jax.ShapeDtypeStruct((N_NODES, D), jnp.float32),
)


@jax.jit
def kernel(x, edge_index, adj_values, W1, b1, W2, b2, W3, b3):
    npad = E_PAD - N_EDGES
    src = jnp.pad(edge_index[0].astype(jnp.int32), (0, npad)
                  ).reshape(N_ROWS_IDX, CHUNK)
    dst = jnp.pad(edge_index[1].astype(jnp.int32), (0, npad),
                  constant_values=N_NODES).reshape(N_ROWS_IDX, CHUNK)
    val = lax.bitcast_convert_type(jnp.pad(adj_values, (0, npad)),
                                   jnp.int32).reshape(N_ROWS_IDX, CHUNK)
    pidx = jnp.stack([src, dst, val], axis=1)
    b1r = b1.reshape(1, D)
    b2r = b2.reshape(1, D)
    b3r = b3.reshape(1, D)

    sup = _mm_first(x, W1)
    p = _spmm(sup, pidx)
    sup = _mm_mid(p, b1r, W2)
    p = _spmm(sup, pidx)
    sup = _mm_mid(p, b2r, W3)
    p = _spmm(sup, pidx)
    return _combine(p, b3r)
